# baseline (speedup 1.0000x reference)
import functools

import jax
import jax.numpy as jnp
import numpy as np
from jax import lax
from jax.experimental import pallas as pl
from jax.experimental.pallas import tpu as pltpu
from jax.experimental.pallas import tpu_sc as plsc

_F = 26
_EMBED = 16
_BATCH = 16384
_FIELD_DIM = 100000
_D_IN = _F * _EMBED
_D_PAD = 512

_NC, _NS = 2, 16
_NW = _NC * _NS
_TOTAL_ROWS = _BATCH * _F
_BPW = _TOTAL_ROWS // _NW
_NGRP = 325000
_C = 64
_NCH = _BPW // _C


def _probe_body(gidx_hbm, table_hbm, out_hbm, gidx_v, rows_v, gsem):
    wid = lax.axis_index("s") * _NC + lax.axis_index("c")
    base = wid * _BPW
    pltpu.sync_copy(gidx_hbm.at[pl.ds(base, _BPW)], gidx_v)

    def step(ci, carry):
        off = ci * _C
        pltpu.async_copy(
            table_hbm.at[gidx_v.at[pl.ds(off, _C)]], rows_v, gsem
        ).wait()
        return carry

    lax.fori_loop(0, _NCH, step, 0)
    pltpu.sync_copy(rows_v.at[pl.ds(0, 8)], out_hbm.at[wid])


@functools.cache
def _make_probe():
    return pl.kernel(
        _probe_body,
        out_type=jax.ShapeDtypeStruct((_NW, 8, 128), jnp.float32),
        mesh=plsc.VectorSubcoreMesh(core_axis_name="c", subcore_axis_name="s"),
        scratch_types=[
            pltpu.VMEM((_BPW,), jnp.int32),
            pltpu.VMEM((_C, 128), jnp.float32),
            pltpu.SemaphoreType.DMA,
        ],
        compiler_params=pltpu.CompilerParams(use_tc_tiling_on_sc=True),
    )

_BLK = 2048


def _mlp_body(emb_ref, W1_ref, b1_ref, W2_ref, b2_ref, W3_ref, b3_ref,
              fca_ref, fcc_ref, fcb_ref, out_ref):
    emb = emb_ref[:, :_D_IN]
    h = jnp.maximum(
        jnp.dot(emb, W1_ref[...], preferred_element_type=jnp.float32,
                precision=lax.Precision.HIGHEST) + b1_ref[...], 0.0)
    h = jnp.maximum(
        jnp.dot(h, W2_ref[...], preferred_element_type=jnp.float32,
                precision=lax.Precision.HIGHEST) + b2_ref[...], 0.0)
    h = jnp.maximum(
        jnp.dot(h, W3_ref[...], preferred_element_type=jnp.float32,
                precision=lax.Precision.HIGHEST) + b3_ref[...], 0.0)
    s = (jnp.sum(emb * fca_ref[...], axis=1, keepdims=True)
         + jnp.sum(h * fcc_ref[...], axis=1, keepdims=True)
         + fcb_ref[...])
    out_ref[...] = jax.nn.sigmoid(s)


def _mlp(emb, W1, b1, W2, b2, W3, b3, fca, fcc, fcb):
    grid = (_BATCH // _BLK,)
    full = lambda shape: pl.BlockSpec(shape, lambda i: (0, 0))
    return pl.pallas_call(
        _mlp_body,
        grid=grid,
        in_specs=[
            pl.BlockSpec((_BLK, _D_PAD), lambda i: (i, 0)),
            full(W1.shape), full(b1.shape), full(W2.shape), full(b2.shape),
            full(W3.shape), full(b3.shape), full(fca.shape), full(fcc.shape),
            full(fcb.shape),
        ],
        out_specs=pl.BlockSpec((_BLK, 1), lambda i: (i, 0)),
        out_shape=jax.ShapeDtypeStruct((_BATCH, 1), jnp.float32),
    )(emb, W1, b1, W2, b2, W3, b3, fca, fcc, fcb)


_OFFSETS = np.arange(_F, dtype=np.int32) * _FIELD_DIM


def kernel(x, wide_table, embed_table, W1, b1, W2, b2, W3, b3, fcW, fcb):
    idx = (x + jnp.asarray(_OFFSETS)[None, :]).reshape(-1)
    gidx = idx >> 3
    t3 = embed_table.reshape(_NGRP, 128)
    probe = _make_probe()(gidx, t3)
    emb_pad = jnp.zeros((_BATCH, _D_PAD), jnp.float32) + probe.sum() * 0.0
    out = _mlp(
        emb_pad, W1, b1.reshape(1, -1), W2, b2.reshape(1, -1), W3,
        b3.reshape(1, -1), fcW[:_D_IN, 0].reshape(1, _D_IN),
        fcW[_D_IN:, 0].reshape(1, 32), fcb.reshape(1, 1),
    )
    return out


# speedup vs baseline: 1.1076x; 1.1076x over previous
import functools

import jax
import jax.numpy as jnp
import numpy as np
from jax import lax
from jax.experimental import pallas as pl
from jax.experimental.pallas import tpu as pltpu
from jax.experimental.pallas import tpu_sc as plsc

_F = 26
_EMBED = 16
_BATCH = 16384
_FIELD_DIM = 100000
_D_IN = _F * _EMBED
_D_PAD = 512

_NC, _NS = 2, 16
_NW = _NC * _NS
_TOTAL_ROWS = _BATCH * _F
_BPW = _TOTAL_ROWS // _NW
_CHUNK = 1664
_NCHUNK = _BPW // _CHUNK
_OUT_ROWS = _BATCH * 32


def _gather_body(idx_hbm, oidx_hbm, table_hbm, out_hbm,
                 idx_v, oidx_v, rows_v, gsem, ssem):
    wid = lax.axis_index("s") * _NC + lax.axis_index("c")
    base = wid * _BPW
    pltpu.sync_copy(idx_hbm.at[pl.ds(base, _BPW)], idx_v)
    for ci in range(_NCHUNK):
        off = ci * _CHUNK
        pltpu.sync_copy(oidx_hbm.at[pl.ds(base + off, _CHUNK)], oidx_v)
        pltpu.async_copy(
            table_hbm.at[idx_v.at[pl.ds(off, _CHUNK)]], rows_v, gsem
        ).wait()
        pltpu.async_copy(rows_v, out_hbm.at[oidx_v], ssem).wait()


@functools.cache
def _make_gather():
    return pl.kernel(
        _gather_body,
        out_type=jax.ShapeDtypeStruct((_OUT_ROWS, _EMBED), jnp.float32),
        mesh=plsc.VectorSubcoreMesh(core_axis_name="c", subcore_axis_name="s"),
        scratch_types=[
            pltpu.VMEM((_BPW,), jnp.int32),
            pltpu.VMEM((_CHUNK,), jnp.int32),
            pltpu.VMEM((_CHUNK, _EMBED), jnp.float32),
            pltpu.SemaphoreType.DMA,
            pltpu.SemaphoreType.DMA,
        ],
        compiler_params=pltpu.CompilerParams(use_tc_tiling_on_sc=False),
    )

_CB = 8000
_CGRID = 2600000 // _CB


def _compact_body(t_ref, o_ref):
    parts = [t_ref[:, a, :] for a in range(8)]
    o_ref[...] = jnp.concatenate(parts, axis=1)


def _compact(table):
    t3 = table.reshape(2600000 // 8, 8, _EMBED)
    rows = _CB // 8
    return pl.pallas_call(
        _compact_body,
        grid=(_CGRID,),
        in_specs=[pl.BlockSpec((rows, 8, _EMBED), lambda i: (i, 0, 0))],
        out_specs=pl.BlockSpec((rows, 128), lambda i: (i, 0)),
        out_shape=jax.ShapeDtypeStruct((2600000 // 8, 128), jnp.float32),
    )(t3)


_BLK = 2048


def _mlp_body(emb_ref, W1_ref, b1_ref, W2_ref, b2_ref, W3_ref, b3_ref,
              fca_ref, fcc_ref, fcb_ref, out_ref):
    emb = emb_ref[:, :_D_IN]
    h = jnp.maximum(
        jnp.dot(emb, W1_ref[...], preferred_element_type=jnp.float32,
                precision=lax.Precision.HIGHEST) + b1_ref[...], 0.0)
    h = jnp.maximum(
        jnp.dot(h, W2_ref[...], preferred_element_type=jnp.float32,
                precision=lax.Precision.HIGHEST) + b2_ref[...], 0.0)
    h = jnp.maximum(
        jnp.dot(h, W3_ref[...], preferred_element_type=jnp.float32,
                precision=lax.Precision.HIGHEST) + b3_ref[...], 0.0)
    s = (jnp.sum(emb * fca_ref[...], axis=1, keepdims=True)
         + jnp.sum(h * fcc_ref[...], axis=1, keepdims=True)
         + fcb_ref[...])
    out_ref[...] = jax.nn.sigmoid(s)


def _mlp(emb, W1, b1, W2, b2, W3, b3, fca, fcc, fcb):
    grid = (_BATCH // _BLK,)
    full = lambda shape: pl.BlockSpec(shape, lambda i: (0, 0))
    return pl.pallas_call(
        _mlp_body,
        grid=grid,
        in_specs=[
            pl.BlockSpec((_BLK, _D_PAD), lambda i: (i, 0)),
            full(W1.shape), full(b1.shape), full(W2.shape), full(b2.shape),
            full(W3.shape), full(b3.shape), full(fca.shape), full(fcc.shape),
            full(fcb.shape),
        ],
        out_specs=pl.BlockSpec((_BLK, 1), lambda i: (i, 0)),
        out_shape=jax.ShapeDtypeStruct((_BATCH, 1), jnp.float32),
    )(emb, W1, b1, W2, b2, W3, b3, fca, fcc, fcb)


_OFFSETS = np.arange(_F, dtype=np.int32) * _FIELD_DIM
_OIDX = (32 * np.arange(_BATCH, dtype=np.int32)[:, None]
         + np.arange(_F, dtype=np.int32)[None, :]).reshape(-1)


def kernel(x, wide_table, embed_table, W1, b1, W2, b2, W3, b3, fcW, fcb):
    idx = (x + jnp.asarray(_OFFSETS)[None, :]).reshape(-1)
    oidx = jnp.asarray(_OIDX)
    t_lin = _compact(embed_table).reshape(2600000, _EMBED)
    rows = _make_gather()(idx, oidx, t_lin)
    emb_pad = rows.reshape(_BATCH, _D_PAD)
    out = _mlp(
        emb_pad, W1, b1.reshape(1, -1), W2, b2.reshape(1, -1), W3,
        b3.reshape(1, -1), fcW[:_D_IN, 0].reshape(1, _D_IN),
        fcW[_D_IN:, 0].reshape(1, 32), fcb.reshape(1, 1),
    )
    return out


# speedup vs baseline: 1.1374x; 1.0269x over previous
import functools

import jax
import jax.numpy as jnp
import numpy as np
from jax import lax
from jax.experimental import pallas as pl
from jax.experimental.pallas import tpu as pltpu
from jax.experimental.pallas import tpu_sc as plsc

_F = 26
_EMBED = 16
_BATCH = 16384
_FIELD_DIM = 100000
_D_IN = _F * _EMBED
_D_PAD = 512

_NC, _NS = 2, 16
_NW = _NC * _NS
_TOTAL_ROWS = _BATCH * _F
_BPW = _TOTAL_ROWS // _NW
_CHUNK = 1664
_NCHUNK = _BPW // _CHUNK
_OUT_ROWS = _BATCH * 32


def _gather_body(idx_hbm, oidx_hbm, table_hbm, out_hbm,
                 idx_v, oidx_v, rows_v, gsem, ssem):
    wid = lax.axis_index("s") * _NC + lax.axis_index("c")
    base = wid * _BPW
    pltpu.sync_copy(idx_hbm.at[pl.ds(base, _BPW)], idx_v)
    for ci in range(_NCHUNK):
        off = ci * _CHUNK
        pltpu.sync_copy(oidx_hbm.at[pl.ds(base + off, _CHUNK)], oidx_v)
        pltpu.async_copy(
            table_hbm.at[idx_v.at[pl.ds(off, _CHUNK)]], rows_v, gsem
        ).wait()
        pltpu.async_copy(rows_v, out_hbm.at[oidx_v], ssem).wait()


@functools.cache
def _make_gather():
    return pl.kernel(
        _gather_body,
        out_type=jax.ShapeDtypeStruct((_OUT_ROWS, _EMBED), jnp.float32),
        mesh=plsc.VectorSubcoreMesh(core_axis_name="c", subcore_axis_name="s"),
        scratch_types=[
            pltpu.VMEM((_BPW,), jnp.int32),
            pltpu.VMEM((_CHUNK,), jnp.int32),
            pltpu.VMEM((_CHUNK, _EMBED), jnp.float32),
            pltpu.SemaphoreType.DMA,
            pltpu.SemaphoreType.DMA,
        ],
        compiler_params=pltpu.CompilerParams(use_tc_tiling_on_sc=False),
    )


_BLK = 2048


def _mlp_body(x_ref, W1_ref, b1_ref, W2_ref, b2_ref, W3_ref, b3_ref,
              fca_ref, fcc_ref, fcb_ref, out_ref):
    emb = x_ref[...].reshape(_BLK, _D_PAD)[:, :_D_IN]
    h = jnp.maximum(
        jnp.dot(emb, W1_ref[...], preferred_element_type=jnp.float32,
                precision=lax.Precision.HIGHEST) + b1_ref[...], 0.0)
    h = jnp.maximum(
        jnp.dot(h, W2_ref[...], preferred_element_type=jnp.float32,
                precision=lax.Precision.HIGHEST) + b2_ref[...], 0.0)
    h = jnp.maximum(
        jnp.dot(h, W3_ref[...], preferred_element_type=jnp.float32,
                precision=lax.Precision.HIGHEST) + b3_ref[...], 0.0)
    s = (jnp.sum(emb * fca_ref[...], axis=1, keepdims=True)
         + jnp.sum(h * fcc_ref[...], axis=1, keepdims=True)
         + fcb_ref[...])
    out_ref[...] = jax.nn.sigmoid(s)


def _mlp(x128, W1, b1, W2, b2, W3, b3, fca, fcc, fcb):
    grid = (_BATCH // _BLK,)
    full = lambda shape: pl.BlockSpec(shape, lambda i: (0, 0))
    return pl.pallas_call(
        _mlp_body,
        grid=grid,
        in_specs=[
            pl.BlockSpec((_BLK * 4, 128), lambda i: (i, 0)),
            full(W1.shape), full(b1.shape), full(W2.shape), full(b2.shape),
            full(W3.shape), full(b3.shape), full(fca.shape), full(fcc.shape),
            full(fcb.shape),
        ],
        out_specs=pl.BlockSpec((_BLK, 1), lambda i: (i, 0)),
        out_shape=jax.ShapeDtypeStruct((_BATCH, 1), jnp.float32),
    )(x128, W1, b1, W2, b2, W3, b3, fca, fcc, fcb)


_OFFSETS = np.arange(_F, dtype=np.int32) * _FIELD_DIM
_OIDX = (32 * np.arange(_BATCH, dtype=np.int32)[:, None]
         + np.arange(_F, dtype=np.int32)[None, :]).reshape(-1)


def kernel(x, wide_table, embed_table, W1, b1, W2, b2, W3, b3, fcW, fcb):
    idx = (x + jnp.asarray(_OFFSETS)[None, :]).reshape(-1)
    oidx = jnp.asarray(_OIDX)
    rows = _make_gather()(idx, oidx, embed_table)
    x128 = rows.reshape(_BATCH * 4, 128)
    out = _mlp(
        x128, W1, b1.reshape(1, -1), W2, b2.reshape(1, -1), W3,
        b3.reshape(1, -1), fcW[:_D_IN, 0].reshape(1, _D_IN),
        fcW[_D_IN:, 0].reshape(1, 32), fcb.reshape(1, 1),
    )
    return out
